# baseline (speedup 1.0000x reference)
import functools
import math

import jax
import jax.numpy as jnp
from jax import lax
from jax.experimental import pallas as pl
from jax.experimental.pallas import tpu as pltpu
from jax.experimental.pallas import tpu_sc as plsc

_VOCAB = 100000
_PAD_IDX = 0
_SMOOTH = 0.1
_CONF = 1.0 - _SMOOTH
_N_TOK = 2048
_SVAL = _SMOOTH / (_VOCAB - 2)
_E_TERM = (_VOCAB - 2) * _SVAL * math.log(_SVAL) + _CONF * math.log(_CONF)

_RB = 256
_VB = 12800
_RGRID = _N_TOK // _RB
_VGRID = -(-_VOCAB // _VB)

_NW = 32
_RW = _N_TOK // _NW
_LANES = 16


def _rowsum_body(y_ref, out_ref):
    j = pl.program_id(1)
    cols = j * _VB + lax.broadcasted_iota(jnp.int32, (_RB, _VB), 1)
    yb = jnp.where(cols < _VOCAB, y_ref[...], 0.0)
    part = jnp.sum(yb, axis=1, keepdims=True)

    @pl.when(j == 0)
    def _init():
        out_ref[...] = part

    @pl.when(j != 0)
    def _acc():
        out_ref[...] += part


def _rowsums(y):
    return pl.pallas_call(
        _rowsum_body,
        grid=(_RGRID, _VGRID),
        in_specs=[pl.BlockSpec((_RB, _VB), lambda i, j: (i, j))],
        out_specs=pl.BlockSpec((_RB, 1), lambda i, j: (i, 0)),
        out_shape=jax.ShapeDtypeStruct((_N_TOK, 1), jnp.float32),
    )(y)


def _sc_body(yflat_hbm, tgt_hbm, rs_hbm, out_hbm,
             tgt_v, idx_v, gat_v, rs_v, acc_v, sem):
    wid = lax.axis_index("s") * 2 + lax.axis_index("c")
    base = wid * _RW
    pltpu.sync_copy(tgt_hbm.at[pl.ds(base, _RW)], tgt_v)
    pltpu.sync_copy(rs_hbm.at[pl.ds(base, _RW)], rs_v)
    for k in range(_RW // _LANES):
        rows16 = lax.iota(jnp.int32, _LANES) + (base + k * _LANES)
        t16 = tgt_v[pl.ds(k * _LANES, _LANES)]
        idx_v[pl.ds(k * _LANES, _LANES)] = rows16 * _VOCAB + t16
        idx_v[pl.ds(_RW + k * _LANES, _LANES)] = rows16 * _VOCAB
    pltpu.async_copy(yflat_hbm.at[idx_v], gat_v, sem).wait()
    acc = jnp.zeros((_LANES,), jnp.float32)
    for k in range(_RW // _LANES):
        t16 = tgt_v[pl.ds(k * _LANES, _LANES)]
        yt = gat_v[pl.ds(k * _LANES, _LANES)]
        y0 = gat_v[pl.ds(_RW + k * _LANES, _LANES)]
        rs = rs_v[pl.ds(k * _LANES, _LANES)]
        contrib = (_E_TERM
                   - _SVAL * (rs - y0 - yt)
                   - _CONF * yt)
        acc = acc + jnp.where(t16 != _PAD_IDX, contrib, 0.0)
    acc_v[...] = acc
    pltpu.sync_copy(acc_v, out_hbm.at[pl.ds(wid * _LANES, _LANES)])


def _sc_combine(y_flat, target, rowsums):
    mesh = plsc.VectorSubcoreMesh(core_axis_name="c", subcore_axis_name="s")
    fn = pl.kernel(
        _sc_body,
        out_type=jax.ShapeDtypeStruct((_NW * _LANES,), jnp.float32),
        mesh=mesh,
        scratch_types=[
            pltpu.VMEM((_RW,), jnp.int32),
            pltpu.VMEM((2 * _RW,), jnp.int32),
            pltpu.VMEM((2 * _RW,), jnp.float32),
            pltpu.VMEM((_RW,), jnp.float32),
            pltpu.VMEM((_LANES,), jnp.float32),
            pltpu.SemaphoreType.DMA,
        ],
    )
    return fn(y_flat, target, rowsums)


def kernel(y, target):
    rowsums = _rowsums(y)
    sc_out = _sc_combine(y.reshape(-1), target.astype(jnp.int32),
                         rowsums.reshape(-1))
    return jnp.sum(sc_out)


# speedup vs baseline: 2.1794x; 2.1794x over previous
import math

import jax
import jax.numpy as jnp
from jax import lax
from jax.experimental import pallas as pl
from jax.experimental.pallas import tpu as pltpu
from jax.experimental.pallas import tpu_sc as plsc

_VOCAB = 100000
_PAD_IDX = 0
_SMOOTH = 0.1
_CONF = 1.0 - _SMOOTH
_N_TOK = 2048
_SVAL = _SMOOTH / (_VOCAB - 2)
_E_TERM = (_VOCAB - 2) * _SVAL * math.log(_SVAL) + _CONF * math.log(_CONF)

_RB = 256
_VB = 12800
_RGRID = _N_TOK // _RB
_VGRID = -(-_VOCAB // _VB)

_NW = 32
_RW = _N_TOK // _NW
_LANES = 16


def _rowsum_body(y_ref, out_ref):
    j = pl.program_id(1)

    @pl.when(j == 0)
    def _init():
        out_ref[...] = jnp.sum(y_ref[...], axis=1, keepdims=True)

    @pl.when(jnp.logical_and(j > 0, j < _VGRID - 1))
    def _acc():
        out_ref[...] += jnp.sum(y_ref[...], axis=1, keepdims=True)

    @pl.when(j == _VGRID - 1)
    def _acc_tail():
        cols = j * _VB + lax.broadcasted_iota(jnp.int32, (_RB, _VB), 1)
        yb = jnp.where(cols < _VOCAB, y_ref[...], 0.0)
        out_ref[...] += jnp.sum(yb, axis=1, keepdims=True)


def _rowsums(y):
    return pl.pallas_call(
        _rowsum_body,
        grid=(_RGRID, _VGRID),
        in_specs=[pl.BlockSpec((_RB, _VB), lambda i, j: (i, j))],
        out_specs=pl.BlockSpec((_RB, 1), lambda i, j: (i, 0)),
        out_shape=jax.ShapeDtypeStruct((_N_TOK, 1), jnp.float32),
    )(y)


_TILE_S = 8
_TILE_L = 128
_CB_MAX = _VOCAB - _TILE_L


def _sc_body(y_hbm, tgt_hbm, rs_hbm, out_hbm,
             tgt_v, rs_v, buf_t, buf_0, acc_v, sem, sem2):
    wid = lax.axis_index("s") * 2 + lax.axis_index("c")
    base = wid * _RW
    pltpu.sync_copy(tgt_hbm.at[pl.ds(base, _RW)], tgt_v)
    pltpu.sync_copy(rs_hbm.at[pl.ds(base, _RW)], rs_v)
    col0 = pltpu.async_copy(
        y_hbm.at[pl.ds(base, _RW), pl.ds(0, _TILE_L)], buf_0, sem2)
    iota16 = lax.iota(jnp.int32, _LANES)
    copies = []
    for r in range(_RW):
        t16 = tgt_v[pl.ds((r // _LANES) * _LANES, _LANES)]
        t = jnp.sum(jnp.where(iota16 == (r % _LANES), t16, 0), axis=0)
        cb = pl.multiple_of((t // _TILE_L) * _TILE_L, _TILE_L)
        rg = pl.multiple_of(base + (r // _TILE_S) * _TILE_S, _TILE_S)
        copies.append(pltpu.async_copy(
            y_hbm.at[pl.ds(rg, _TILE_S), pl.ds(cb, _TILE_L)],
            buf_t.at[r], sem))
    col0.wait()
    for c in copies:
        c.wait()
    acc = jnp.zeros((_LANES,), jnp.float32)
    zeros16 = jnp.zeros((_LANES,), jnp.int32)
    for k in range(_RW // _LANES):
        t16 = tgt_v[pl.ds(k * _LANES, _LANES)]
        rows16 = lax.iota(jnp.int32, _LANES) + (k * _LANES)
        sub16 = lax.rem(rows16, _TILE_S)
        lanes16 = lax.rem(t16, _TILE_L)
        yt = plsc.load_gather(buf_t, [rows16, sub16, lanes16])
        y0 = plsc.load_gather(buf_0, [rows16, zeros16])
        rs = rs_v[pl.ds(k * _LANES, _LANES)]
        contrib = (_E_TERM
                   - _SVAL * (rs - y0 - yt)
                   - _CONF * yt)
        acc = acc + jnp.where(t16 != _PAD_IDX, contrib, 0.0)
    acc_v[...] = acc
    pltpu.sync_copy(acc_v, out_hbm.at[pl.ds(wid * _LANES, _LANES)])


def _sc_combine(y, target, rowsums):
    mesh = plsc.VectorSubcoreMesh(core_axis_name="c", subcore_axis_name="s")
    fn = pl.kernel(
        _sc_body,
        out_type=jax.ShapeDtypeStruct((_NW * _LANES,), jnp.float32),
        mesh=mesh,
        compiler_params=pltpu.CompilerParams(needs_layout_passes=False),
        scratch_types=[
            pltpu.VMEM((_RW,), jnp.int32),
            pltpu.VMEM((_RW,), jnp.float32),
            pltpu.VMEM((_RW, _TILE_S, _TILE_L), jnp.float32),
            pltpu.VMEM((_RW, _TILE_L), jnp.float32),
            pltpu.VMEM((_LANES,), jnp.float32),
            pltpu.SemaphoreType.DMA,
            pltpu.SemaphoreType.DMA,
        ],
    )
    return fn(y, target, rowsums)


def kernel(y, target):
    rowsums = _rowsums(y)
    sc_out = _sc_combine(y, target.astype(jnp.int32), rowsums.reshape(-1))
    return jnp.sum(sc_out)
